# baseline (speedup 1.0000x reference)
import jax
import jax.numpy as jnp
from jax import lax
from jax.experimental import pallas as pl
from jax.experimental.pallas import tpu as pltpu

N = 50000
E = 800000
B = 256
L = 1000
NT = 25
T = 2048
NP = NT * T
C = 256
E2 = E + N
EPMAX = ((E2 + NT * C + C - 1) // C) * C
EPC = EPMAX // C


def _mm_scale_body(h_ref, w_ref, d_ref, y_ref):
    y_ref[...] = d_ref[...] * jnp.dot(h_ref[...], w_ref[...],
                                      preferred_element_type=jnp.float32)


def _mm_scale(h, w, dinv):
    K = h.shape[1]
    Fp = w.shape[1]
    return pl.pallas_call(
        _mm_scale_body,
        grid=(NP // 512,),
        in_specs=[
            pl.BlockSpec((512, K), lambda i: (i, 0)),
            pl.BlockSpec((K, Fp), lambda i: (0, 0)),
            pl.BlockSpec((512, 1), lambda i: (i, 0)),
        ],
        out_specs=pl.BlockSpec((512, Fp), lambda i: (i, 0)),
        out_shape=jax.ShapeDtypeStruct((NP, Fp), jnp.float32),
    )(h, w, dinv)


def _finish_body(a_ref, d_ref, b_ref, h_ref):
    h_ref[...] = jnp.maximum(d_ref[...] * a_ref[...] + b_ref[...], 0.0)


def _finish(agg, dinv, b):
    Fp = agg.shape[1]
    return pl.pallas_call(
        _finish_body,
        grid=(NP // 512,),
        in_specs=[
            pl.BlockSpec((512, Fp), lambda i: (i, 0)),
            pl.BlockSpec((512, 1), lambda i: (i, 0)),
            pl.BlockSpec((1, Fp), lambda i: (0, 0)),
        ],
        out_specs=pl.BlockSpec((512, Fp), lambda i: (i, 0)),
        out_shape=jax.ShapeDtypeStruct((NP, Fp), jnp.float32),
    )(agg, dinv, b)


def _shift_sum(u0, u1, u2):
    z = jnp.zeros((1, u0.shape[1]), jnp.float32)
    return (jnp.concatenate([z, u0[:-1]], axis=0)
            + u1
            + jnp.concatenate([u2[1:], z], axis=0))


def _cnn_body(tgt_ref, emb_ref, w1_ref, b1_ref, w2_ref, b2_ref,
              w3_ref, b3_ref, p1_ref, p2_ref, p3_ref):
    tgt = tgt_ref[0, 0, :]
    oh = (tgt[:, None] == lax.broadcasted_iota(jnp.int32, (L, 32), 1))
    x = jnp.dot(oh.astype(jnp.float32), emb_ref[...],
                preferred_element_type=jnp.float32)

    def conv(xin, w_ref, b_ref):
        u = [jnp.dot(xin, w_ref[k], preferred_element_type=jnp.float32)
             for k in range(3)]
        y = _shift_sum(u[0], u[1], u[2]) + b_ref[...]
        return jnp.maximum(y, 0.0)

    a1 = conv(x, w1_ref, b1_ref)
    a2 = conv(a1, w2_ref, b2_ref)
    a3 = conv(a2, w3_ref, b3_ref)
    p1_ref[0, 0, :] = jnp.max(a1, axis=0)
    p2_ref[0, 0, :] = jnp.max(a2, axis=0)
    p3_ref[0, 0, :] = jnp.max(a3, axis=0)


def _cnn(target3, embp, w1t, b1, w2t, b2, w3t, b3):
    out3 = jax.ShapeDtypeStruct((B, 1, 128), jnp.float32)
    po = pl.BlockSpec((1, 1, 128), lambda i: (i, 0, 0))
    return pl.pallas_call(
        _cnn_body,
        grid=(B,),
        in_specs=[
            pl.BlockSpec((1, 1, L), lambda i: (i, 0, 0)),
            pl.BlockSpec((32, 256), lambda i: (0, 0)),
            pl.BlockSpec((3, 256, 128), lambda i: (0, 0, 0)),
            pl.BlockSpec((1, 128), lambda i: (0, 0)),
            pl.BlockSpec((3, 128, 128), lambda i: (0, 0, 0)),
            pl.BlockSpec((1, 128), lambda i: (0, 0)),
            pl.BlockSpec((3, 128, 128), lambda i: (0, 0, 0)),
            pl.BlockSpec((1, 128), lambda i: (0, 0)),
        ],
        out_specs=[po, po, po],
        out_shape=[out3, out3, out3],
    )(target3, embp, w1t, b1, w2t, b2, w3t, b3)


def _head_body(xg_ref, xt_ref, wg1_ref, bg1_ref, wg2_ref, bg2_ref,
               ws1_ref, bs1_ref, ws2_ref, bs2_ref, wf1_ref, bf1_ref,
               wf2_ref, bf2_ref, wo_ref, bo_ref, o_ref):
    dot = lambda a, b: jnp.dot(a, b, preferred_element_type=jnp.float32)
    xg = jnp.maximum(dot(xg_ref[...], wg1_ref[...]) + bg1_ref[...], 0.0)
    xg = dot(xg, wg2_ref[...]) + bg2_ref[...]
    xt = dot(xt_ref[...], ws1_ref[...]) + bs1_ref[...]
    xt = dot(xt, ws2_ref[...]) + bs2_ref[...]
    xc = jnp.concatenate([xg, xt], axis=1)
    xc = jnp.maximum(dot(xc, wf1_ref[...]) + bf1_ref[...], 0.0)
    xc = jnp.maximum(dot(xc, wf2_ref[...]) + bf2_ref[...], 0.0)
    o_ref[...] = dot(xc, wo_ref[...]) + bo_ref[...]


def _head(xg, xt, Wg1, bg1, Wg2, bg2, Ws1, bs1, Ws2, bs2,
          Wf1, bf1, Wf2, bf2, Wo, bo):
    args = (xg, xt, Wg1, bg1[None], Wg2, bg2[None], Ws1, bs1[None],
            Ws2, bs2[None], Wf1, bf1[None], Wf2, bf2[None], Wo, bo[None])
    return pl.pallas_call(
        _head_body,
        out_shape=jax.ShapeDtypeStruct((B, 1), jnp.float32),
    )(*args)


def kernel(x, edge_index, batch, target, hidden, cell, W1, b1, W2, b2, W3, b3,
           Wg1, bg1, Wg2, bg2, emb, Wc1, bc1, Wc2, bc2, Wc3, bc3,
           Ws1, bs1, Ws2, bs2, Wf1, bf1, Wf2, bf2, Wo, bo):
    f32 = jnp.float32
    loop = jnp.arange(N, dtype=jnp.int32)
    src2 = jnp.concatenate([edge_index[0].astype(jnp.int32), loop])
    dst2 = jnp.concatenate([edge_index[1].astype(jnp.int32), loop])

    order = jnp.argsort(dst2)
    dst_s = dst2[order]
    src_s = src2[order]

    bounds = jnp.searchsorted(dst_s, jnp.arange(N + 1, dtype=jnp.int32))
    deg = (bounds[1:] - bounds[:-1]).astype(f32)
    dinv = lax.rsqrt(deg)
    dinvp = jnp.concatenate([dinv, jnp.ones((NP - N,), f32)])[:, None]

    def pad2(w, r, c):
        return jnp.pad(w, ((0, r - w.shape[0]), (0, c - w.shape[1])))

    xp = jnp.pad(x, ((0, NP - N), (0, 0)))
    W1p, b1p = pad2(W1, 78, 128), jnp.pad(b1, (0, 50))[None]
    W2p, b2p = pad2(W2, 128, 256), jnp.pad(b2, (0, 100))[None]
    W3p, b3p = pad2(W3, 256, 384), jnp.pad(b3, (0, 72))[None]

    def gcn_layer(h, Wp, bp):
        y = _mm_scale(h, Wp, dinvp)
        agg = jax.ops.segment_sum(y[src_s], dst_s, num_segments=NP,
                                  indices_are_sorted=True)
        return _finish(agg, dinvp, bp)

    h1 = gcn_layer(xp, W1p, b1p)
    h2 = gcn_layer(h1, W2p, b2p)
    h3 = gcn_layer(h2, W3p, b3p)

    hc = jnp.concatenate([h1[:N, :78], h2[:N, :156], h3[:N, :312]], axis=1)
    xg = jax.ops.segment_max(hc, batch, num_segments=B,
                             indices_are_sorted=True)

    embp = jnp.pad(emb, ((0, 6), (0, 0)))
    p1, p2, p3 = _cnn(target.astype(jnp.int32)[:, None, :], embp,
                      jnp.transpose(Wc1, (2, 1, 0)), bc1[None],
                      jnp.transpose(Wc2, (2, 1, 0)), bc2[None],
                      jnp.transpose(Wc3, (2, 1, 0)), bc3[None])
    xt = jnp.concatenate([p1[:, 0, :], p2[:, 0, :], p3[:, 0, :]], axis=1)

    return _head(xg, xt, Wg1, bg1, Wg2, bg2, Ws1, bs1, Ws2, bs2,
                 Wf1, bf1, Wf2, bf2, Wo, bo)


# speedup vs baseline: 2.4245x; 2.4245x over previous
import jax
import jax.numpy as jnp
from jax import lax
from jax.experimental import pallas as pl
from jax.experimental.pallas import tpu as pltpu

N = 50000
E = 800000
B = 256
L = 1000
NT = 25
T = 2048
NP = NT * T
C = 256
E2 = E + N
EPMAX = ((E2 + NT * C + C - 1) // C) * C
EPC = EPMAX // C


def _mm_scale_body(h_ref, w_ref, d_ref, y_ref):
    y_ref[...] = d_ref[...] * jnp.dot(h_ref[...], w_ref[...],
                                      preferred_element_type=jnp.float32)


def _mm_scale(h, w, dinv):
    K = h.shape[1]
    Fp = w.shape[1]
    return pl.pallas_call(
        _mm_scale_body,
        grid=(NP // 512,),
        in_specs=[
            pl.BlockSpec((512, K), lambda i: (i, 0)),
            pl.BlockSpec((K, Fp), lambda i: (0, 0)),
            pl.BlockSpec((512, 1), lambda i: (i, 0)),
        ],
        out_specs=pl.BlockSpec((512, Fp), lambda i: (i, 0)),
        out_shape=jax.ShapeDtypeStruct((NP, Fp), jnp.float32),
    )(h, w, dinv)


def _finish_body(a_ref, d_ref, b_ref, h_ref):
    h_ref[...] = jnp.maximum(d_ref[...] * a_ref[...] + b_ref[...], 0.0)


def _finish(agg, dinv, b):
    Fp = agg.shape[1]
    return pl.pallas_call(
        _finish_body,
        grid=(NP // 512,),
        in_specs=[
            pl.BlockSpec((512, Fp), lambda i: (i, 0)),
            pl.BlockSpec((512, 1), lambda i: (i, 0)),
            pl.BlockSpec((1, Fp), lambda i: (0, 0)),
        ],
        out_specs=pl.BlockSpec((512, Fp), lambda i: (i, 0)),
        out_shape=jax.ShapeDtypeStruct((NP, Fp), jnp.float32),
    )(agg, dinv, b)


def _shift_sum(u0, u1, u2):
    z = jnp.zeros((1, u0.shape[1]), jnp.float32)
    return (jnp.concatenate([z, u0[:-1]], axis=0)
            + u1
            + jnp.concatenate([u2[1:], z], axis=0))


def _cnn_body(tgt_ref, emb_ref, w1_ref, b1_ref, w2_ref, b2_ref,
              w3_ref, b3_ref, p1_ref, p2_ref, p3_ref):
    tgt = tgt_ref[0, 0, :]
    oh = (tgt[:, None] == lax.broadcasted_iota(jnp.int32, (L, 32), 1))
    x = jnp.dot(oh.astype(jnp.float32), emb_ref[...],
                preferred_element_type=jnp.float32)

    def conv(xin, w_ref, b_ref):
        u = [jnp.dot(xin, w_ref[k], preferred_element_type=jnp.float32)
             for k in range(3)]
        y = _shift_sum(u[0], u[1], u[2]) + b_ref[...]
        return jnp.maximum(y, 0.0)

    a1 = conv(x, w1_ref, b1_ref)
    a2 = conv(a1, w2_ref, b2_ref)
    a3 = conv(a2, w3_ref, b3_ref)
    p1_ref[0, 0, :] = jnp.max(a1, axis=0)
    p2_ref[0, 0, :] = jnp.max(a2, axis=0)
    p3_ref[0, 0, :] = jnp.max(a3, axis=0)


def _cnn(target3, embp, w1t, b1, w2t, b2, w3t, b3):
    out3 = jax.ShapeDtypeStruct((B, 1, 128), jnp.float32)
    po = pl.BlockSpec((1, 1, 128), lambda i: (i, 0, 0))
    return pl.pallas_call(
        _cnn_body,
        grid=(B,),
        in_specs=[
            pl.BlockSpec((1, 1, L), lambda i: (i, 0, 0)),
            pl.BlockSpec((32, 256), lambda i: (0, 0)),
            pl.BlockSpec((3, 256, 128), lambda i: (0, 0, 0)),
            pl.BlockSpec((1, 128), lambda i: (0, 0)),
            pl.BlockSpec((3, 128, 128), lambda i: (0, 0, 0)),
            pl.BlockSpec((1, 128), lambda i: (0, 0)),
            pl.BlockSpec((3, 128, 128), lambda i: (0, 0, 0)),
            pl.BlockSpec((1, 128), lambda i: (0, 0)),
        ],
        out_specs=[po, po, po],
        out_shape=[out3, out3, out3],
    )(target3, embp, w1t, b1, w2t, b2, w3t, b3)


def _head_body(xg_ref, xt_ref, wg1_ref, bg1_ref, wg2_ref, bg2_ref,
               ws1_ref, bs1_ref, ws2_ref, bs2_ref, wf1_ref, bf1_ref,
               wf2_ref, bf2_ref, wo_ref, bo_ref, o_ref):
    dot = lambda a, b: jnp.dot(a, b, preferred_element_type=jnp.float32)
    xg = jnp.maximum(dot(xg_ref[...], wg1_ref[...]) + bg1_ref[...], 0.0)
    xg = dot(xg, wg2_ref[...]) + bg2_ref[...]
    xt = dot(xt_ref[...], ws1_ref[...]) + bs1_ref[...]
    xt = dot(xt, ws2_ref[...]) + bs2_ref[...]
    xc = jnp.concatenate([xg, xt], axis=1)
    xc = jnp.maximum(dot(xc, wf1_ref[...]) + bf1_ref[...], 0.0)
    xc = jnp.maximum(dot(xc, wf2_ref[...]) + bf2_ref[...], 0.0)
    o_ref[...] = dot(xc, wo_ref[...]) + bo_ref[...]


def _head(xg, xt, Wg1, bg1, Wg2, bg2, Ws1, bs1, Ws2, bs2,
          Wf1, bf1, Wf2, bf2, Wo, bo):
    args = (xg, xt, Wg1, bg1[None], Wg2, bg2[None], Ws1, bs1[None],
            Ws2, bs2[None], Wf1, bf1[None], Wf2, bf2[None], Wo, bo[None])
    return pl.pallas_call(
        _head_body,
        out_shape=jax.ShapeDtypeStruct((B, 1), jnp.float32),
    )(*args)


def kernel(x, edge_index, batch, target, hidden, cell, W1, b1, W2, b2, W3, b3,
           Wg1, bg1, Wg2, bg2, emb, Wc1, bc1, Wc2, bc2, Wc3, bc3,
           Ws1, bs1, Ws2, bs2, Wf1, bf1, Wf2, bf2, Wo, bo):
    f32 = jnp.float32
    loop = jnp.arange(N, dtype=jnp.int32)
    src2 = jnp.concatenate([edge_index[0].astype(jnp.int32), loop])
    dst2 = jnp.concatenate([edge_index[1].astype(jnp.int32), loop])

    deg = jnp.zeros((N,), f32).at[dst2].add(1.0)
    dinv = lax.rsqrt(deg)
    dinvp = jnp.concatenate([dinv, jnp.ones((NP - N,), f32)])[:, None]

    xp = jnp.pad(x, ((0, NP - N), (0, 0)))

    def gcn_layer(h, W, b):
        y = _mm_scale(h, W, dinvp)
        agg = jnp.zeros((NP, W.shape[1]), f32).at[dst2].add(y[src2])
        return _finish(agg, dinvp, b[None])

    h1 = gcn_layer(xp, W1, b1)
    h2 = gcn_layer(h1, W2, b2)
    h3 = gcn_layer(h2, W3, b3)

    hc = jnp.concatenate([h1[:N], h2[:N], h3[:N]], axis=1)
    xg = jax.ops.segment_max(hc, batch, num_segments=B,
                             indices_are_sorted=True)

    embp = jnp.pad(emb, ((0, 6), (0, 0)))
    p1, p2, p3 = _cnn(target.astype(jnp.int32)[:, None, :], embp,
                      jnp.transpose(Wc1, (2, 1, 0)), bc1[None],
                      jnp.transpose(Wc2, (2, 1, 0)), bc2[None],
                      jnp.transpose(Wc3, (2, 1, 0)), bc3[None])
    xt = jnp.concatenate([p1[:, 0, :], p2[:, 0, :], p3[:, 0, :]], axis=1)

    return _head(xg, xt, Wg1, bg1, Wg2, bg2, Ws1, bs1, Ws2, bs2,
                 Wf1, bf1, Wf2, bf2, Wo, bo)
